# baseline (speedup 1.0000x reference)
import functools

import jax
import jax.numpy as jnp
from jax import lax
from jax.experimental import pallas as pl
from jax.experimental.pallas import tpu as pltpu
from jax.experimental.pallas import tpu_sc as plsc

NC = 2
NS = 16
NW = NC * NS
LANES = 16
NBUF = 3


def _build_emb_kernel(B, L, D, CH):
  W = L // NW
  NCH = (B * W) // CH
  mesh = plsc.VectorSubcoreMesh(core_axis_name="c", subcore_axis_name="s")

  @functools.partial(
      pl.kernel,
      out_type=[
          jax.ShapeDtypeStruct((B, L, D), jnp.float32),
          jax.ShapeDtypeStruct((B, 1, 1, L), jnp.float32),
      ],
      mesh=mesh,
      scratch_types=[
          pltpu.VMEM((B, W), jnp.int32),
          pltpu.VMEM((W, D), jnp.float32),
          [pltpu.VMEM((CH, D), jnp.float32) for _ in range(NBUF)],
          pltpu.VMEM((B, W), jnp.float32),
          pltpu.SemaphoreType.DMA,
          pltpu.SemaphoreType.DMA,
          pltpu.SemaphoreType.DMA,
          [pltpu.SemaphoreType.DMA for _ in range(NBUF)],
          [pltpu.SemaphoreType.DMA for _ in range(NBUF)],
      ],
  )
  def emb_kernel(ids_hbm, am_hbm, wte_hbm, wpe_hbm, out_hbm, am_out_hbm,
                 idx_v, wpe_v, rows, amb, wsem, asem, isem, gsem, ssem):
    cid = lax.axis_index("c")
    sid = lax.axis_index("s")
    wid = sid * NC + cid
    l0 = wid * W

    idx_cps = [
        pltpu.async_copy(ids_hbm.at[b, pl.ds(l0, W)], idx_v.at[b], isem)
        for b in range(B)
    ]
    am_cps = [
        pltpu.async_copy(am_hbm.at[b, pl.ds(l0, W)], amb.at[b], asem)
        for b in range(B)
    ]
    wpe_cp = pltpu.async_copy(wpe_hbm.at[pl.ds(l0, W)], wpe_v, wsem)
    for cp in idx_cps:
      cp.wait()

    hpc = W // CH

    def start_gather(k):
      b, h = divmod(k, hpc)
      return pltpu.async_copy(
          wte_hbm.at[idx_v.at[b, pl.ds(h * CH, CH)]], rows[k % NBUF],
          gsem[k % NBUF])

    gathers = [None] * NCH
    scatters = [None] * NCH
    for k in range(min(2, NCH)):
      gathers[k] = start_gather(k)
    wpe_cp.wait()

    for k in range(NCH):
      j = k % NBUF
      b, h = divmod(k, hpc)
      gathers[k].wait()

      half = CH // 2

      @plsc.parallel_loop(0, half)
      def add_row_lo(r, rows_ref=rows[j], wbase=h * CH):
        for cc in range(D // LANES):
          sl = pl.ds(cc * LANES, LANES)
          plsc.addupdate(rows_ref.at[r, sl], wpe_v[wbase + r, sl])

      if k + 2 < NCH:
        if k >= 1:
          for cp in scatters[k - 1]:
            cp.wait()
        gathers[k + 2] = start_gather(k + 2)
      lo_cp = pltpu.async_copy(
          rows[j].at[pl.ds(0, half)],
          out_hbm.at[b, pl.ds(l0 + h * CH, half)], ssem[j])

      @plsc.parallel_loop(half, CH)
      def add_row_hi(r, rows_ref=rows[j], wbase=h * CH):
        for cc in range(D // LANES):
          sl = pl.ds(cc * LANES, LANES)
          plsc.addupdate(rows_ref.at[r, sl], wpe_v[wbase + r, sl])

      hi_cp = pltpu.async_copy(
          rows[j].at[pl.ds(half, half)],
          out_hbm.at[b, pl.ds(l0 + h * CH + half, half)], ssem[j])
      scatters[k] = (lo_cp, hi_cp)

    for cp in am_cps:
      cp.wait()
    for b in range(B):
      for i in range(W // LANES):
        sl = pl.ds(i * LANES, LANES)
        amb[b, sl] = (1.0 - amb[b, sl]) * -10000.0
      pltpu.sync_copy(amb.at[b], am_out_hbm.at[b, 0, 0, pl.ds(l0, W)])

    for k in range(max(0, NCH - 3), NCH):
      for cp in scatters[k]:
        cp.wait()

  return emb_kernel


@jax.jit
def kernel(input_ids, attention_mask, wte, wpe):
  B, L = input_ids.shape
  D = wte.shape[1]
  emb = _build_emb_kernel(B, L, D, CH=32)
  hidden, am = emb(input_ids.astype(jnp.int32),
                   attention_mask.astype(jnp.float32),
                   wte.astype(jnp.float32), wpe.astype(jnp.float32))
  return hidden, am


# speedup vs baseline: 1.0852x; 1.0852x over previous
import functools

import jax
import jax.numpy as jnp
from jax import lax
from jax.experimental import pallas as pl
from jax.experimental.pallas import tpu as pltpu
from jax.experimental.pallas import tpu_sc as plsc

NC = 2
NS = 16
NW = NC * NS
LANES = 16


def _build_emb_kernel(B, L, D, CH):
  W = L // NW
  NG = W // CH
  mesh = plsc.VectorSubcoreMesh(core_axis_name="c", subcore_axis_name="s")

  @functools.partial(
      pl.kernel,
      out_type=[
          jax.ShapeDtypeStruct((B, L, D), jnp.float32),
          jax.ShapeDtypeStruct((B, 1, 1, L), jnp.float32),
      ],
      mesh=mesh,
      scratch_types=[
          pltpu.VMEM((B, W), jnp.int32),
          [pltpu.VMEM((CH, D), jnp.float32) for _ in range(2)],
          [[pltpu.VMEM((CH, D), jnp.float32) for _ in range(B)]
           for _ in range(2)],
          pltpu.VMEM((B, W), jnp.float32),
          pltpu.SemaphoreType.DMA,
          pltpu.SemaphoreType.DMA,
          [pltpu.SemaphoreType.DMA for _ in range(2)],
          [[pltpu.SemaphoreType.DMA for _ in range(B)]
           for _ in range(2)],
          [[pltpu.SemaphoreType.DMA for _ in range(B)]
           for _ in range(2)],
      ],
  )
  def emb_kernel(ids_hbm, am_hbm, wte_hbm, wpe_hbm, out_hbm, am_out_hbm,
                 idx_v, wpe_b, rows, amb, asem, isem, wsem, gsem, ssem):
    cid = lax.axis_index("c")
    sid = lax.axis_index("s")
    wid = sid * NC + cid
    l0 = wid * W

    idx_cps = [
        pltpu.async_copy(ids_hbm.at[b, pl.ds(l0, W)], idx_v.at[b], isem)
        for b in range(B)
    ]
    am_cps = [
        pltpu.async_copy(am_hbm.at[b, pl.ds(l0, W)], amb.at[b], asem)
        for b in range(B)
    ]
    for cp in idx_cps:
      cp.wait()

    def start_group(g):
      t = g % 2
      wcp = pltpu.async_copy(
          wpe_hbm.at[pl.ds(l0 + g * CH, CH)], wpe_b[t], wsem[t])
      gcps = [
          pltpu.async_copy(
              wte_hbm.at[idx_v.at[b, pl.ds(g * CH, CH)]], rows[t][b],
              gsem[t][b])
          for b in range(B)
      ]
      return wcp, gcps

    groups = [None] * NG
    scatters = [None] * NG
    groups[0] = start_group(0)

    for g in range(NG):
      t = g % 2
      wcp, gcps = groups[g]
      wcp.wait()
      for cp in gcps:
        cp.wait()
      if g + 1 < NG:
        if g >= 1:
          for cp in scatters[g - 1]:
            cp.wait()
        groups[g + 1] = start_group(g + 1)

      @plsc.parallel_loop(0, CH)
      def add_row(r, t=t):
        for cc in range(D // LANES):
          sl = pl.ds(cc * LANES, LANES)
          wv = wpe_b[t][r, sl]
          for b in range(B):
            plsc.addupdate(rows[t][b].at[r, sl], wv)

      scatters[g] = [
          pltpu.async_copy(
              rows[t][b], out_hbm.at[b, pl.ds(l0 + g * CH, CH)], ssem[t][b])
          for b in range(B)
      ]

    for cp in am_cps:
      cp.wait()
    for b in range(B):
      for i in range(W // LANES):
        sl = pl.ds(i * LANES, LANES)
        amb[b, sl] = (1.0 - amb[b, sl]) * -10000.0
      pltpu.sync_copy(amb.at[b], am_out_hbm.at[b, 0, 0, pl.ds(l0, W)])

    for g in range(max(0, NG - 2), NG):
      for cp in scatters[g]:
        cp.wait()

  return emb_kernel


@jax.jit
def kernel(input_ids, attention_mask, wte, wpe):
  B, L = input_ids.shape
  D = wte.shape[1]
  emb = _build_emb_kernel(B, L, D, CH=16)
  hidden, am = emb(input_ids.astype(jnp.int32),
                   attention_mask.astype(jnp.float32),
                   wte.astype(jnp.float32), wpe.astype(jnp.float32))
  return hidden, am
